# baseline (speedup 1.0000x reference)
import functools

import jax
import jax.numpy as jnp
from jax import lax
from jax.experimental import pallas as pl
from jax.experimental.pallas import tpu as pltpu
from jax.experimental.pallas import tpu_sc as plsc

_NC = 2
_NS = 16
_NW = _NC * _NS


def _mesh():
    return plsc.VectorSubcoreMesh(
        core_axis_name="c", subcore_axis_name="s",
        num_cores=_NC, num_subcores=_NS,
    )


@functools.lru_cache(maxsize=None)
def _make_phase1(V: int, D: int):
    CH = 256
    NFULL = V // CH
    REM = V - NFULL * CH
    PER = NFULL // _NW
    EXTRA = NFULL % _NW
    assert D == 64 and PER % 2 == 0 and REM in (0, 64) and V >= 2 * CH

    @functools.partial(
        pl.kernel,
        out_type=jax.ShapeDtypeStruct((V // 2, 2 * D), jnp.float32),
        mesh=_mesh(),
        scratch_types=[
            [pltpu.VMEM((D, CH), jnp.float32) for _ in range(2)],
            [pltpu.VMEM((CH // 2, 2 * D), jnp.float32) for _ in range(2)],
            [pltpu.SemaphoreType.DMA for _ in range(2)],
            [pltpu.SemaphoreType.DMA for _ in range(2)],
        ],
        compiler_params=pltpu.CompilerParams(needs_layout_passes=False),
    )
    def phase1(wt_hbm, tail_hbm, s_hbm, ins, outs, sem_in, sem_out):
        wid = lax.axis_index("s") * _NC + lax.axis_index("c")
        iota16 = lax.iota(jnp.int32, 16)

        def in_desc(c, b):
            return pltpu.make_async_copy(
                wt_hbm.at[:, pl.ds(c * CH, CH)], ins[b], sem_in[b])

        def out_desc(c, b):
            return pltpu.make_async_copy(
                outs[b], s_hbm.at[pl.ds(c * (CH // 2), CH // 2)], sem_out[b])

        def out_drain(b):
            pltpu.make_async_copy(
                outs[b], s_hbm.at[pl.ds(0, CH // 2)], sem_out[b]).wait()

        def transpose_slab(in_s, out_s, ntok):
            def tg_body(g, carry):
                tb = g * 16
                tvec = tb + iota16
                row = lax.shift_right_logical(tvec, 1)
                colb = lax.bitwise_and(tvec, 1) * D
                for d in range(D):
                    x = in_s[d, pl.ds(tb, 16)]
                    plsc.store_scatter(out_s, [row, colb + d], x)
                return carry
            lax.fori_loop(0, ntok // 16, tg_body, 0)

        nj = PER + jnp.where(wid < EXTRA, 1, 0)
        in_desc(wid, 0).start()

        def pair_body(p, carry):
            for parity in range(2):
                j = 2 * p + parity
                b = parity
                c = wid + _NW * j
                in_desc(c, b).wait()

                @pl.when(j + 1 < nj)
                def _():
                    in_desc(c + _NW, 1 - b).start()

                @pl.when(j >= 2)
                def _():
                    out_drain(b)

                transpose_slab(ins[b], outs[b], CH)
                out_desc(c, b).start()
            return carry

        lax.fori_loop(0, PER // 2, pair_body, 0)

        @pl.when(wid < EXTRA)
        def _():
            c = wid + _NW * PER
            in_desc(c, 0).wait()
            out_drain(0)
            transpose_slab(ins[0], outs[0], CH)
            out_desc(c, 0).start()

        if REM:
            @pl.when(wid == EXTRA)
            def _():
                out_drain(1)
                pltpu.make_async_copy(
                    tail_hbm, outs[1].at[pl.ds(0, REM // 2)],
                    sem_in[1]).start()
                pltpu.make_async_copy(
                    tail_hbm, outs[1].at[pl.ds(0, REM // 2)],
                    sem_in[1]).wait()
                pltpu.make_async_copy(
                    outs[1].at[pl.ds(0, REM // 2)],
                    s_hbm.at[pl.ds((V - REM) // 2, REM // 2)],
                    sem_out[1]).start()

        out_drain(0)
        if REM:
            @pl.when(wid == EXTRA)
            def _():
                pltpu.make_async_copy(
                    outs[1].at[pl.ds(0, REM // 2)],
                    s_hbm.at[pl.ds(0, REM // 2)], sem_out[1]).wait()

            @pl.when(wid != EXTRA)
            def _():
                out_drain(1)
        else:
            out_drain(1)

    return phase1


@functools.lru_cache(maxsize=None)
def _make_phase2(H: int, Bt: int, V: int, D: int):
    TOK = 256
    TPH = Bt // TOK
    TASKS = H * TPH
    PER = TASKS // _NW
    assert D == 64 and Bt % TOK == 0 and TASKS % _NW == 0 and PER % 2 == 0

    @functools.partial(
        pl.kernel,
        out_type=jax.ShapeDtypeStruct((H, D, Bt), jnp.float32),
        mesh=_mesh(),
        scratch_types=[
            [pltpu.VMEM((TOK,), jnp.int32) for _ in range(2)],
            [pltpu.VMEM((TOK,), jnp.int32) for _ in range(2)],
            [pltpu.VMEM((128,), jnp.int32) for _ in range(2)],
            [pltpu.VMEM((128,), jnp.int32) for _ in range(2)],
            [pltpu.VMEM((128, 2 * D), jnp.float32) for _ in range(2)],
            [pltpu.VMEM((128, 2 * D), jnp.float32) for _ in range(2)],
            [pltpu.VMEM((D, TOK), jnp.float32) for _ in range(2)],
            [pltpu.SemaphoreType.DMA for _ in range(2)],
            [pltpu.SemaphoreType.DMA for _ in range(2)],
            [pltpu.SemaphoreType.DMA for _ in range(2)],
        ],
        compiler_params=pltpu.CompilerParams(needs_layout_passes=False),
    )
    def phase2(inpt_hbm, s_hbm, out_hbm, idx, par, qA, qB, gA, gB, ts,
               sem_idx, sem_g, sem_o):
        wid = lax.axis_index("s") * _NC + lax.axis_index("c")
        iota16 = lax.iota(jnp.int32, 16)

        def task_pos(j):
            tau = wid + _NW * j
            return tau // TPH, (tau % TPH) * TOK

        def idx_desc(j, b):
            h, b0 = task_pos(j)
            return pltpu.make_async_copy(
                inpt_hbm.at[h, pl.ds(b0, TOK)], idx[b], sem_idx[b])

        def compute_q(b):
            for g in range(TOK // 16):
                v = idx[b][pl.ds(g * 16, 16)]
                half = qA if g < 8 else qB
                half[b][pl.ds((g % 8) * 16, 16)] = lax.shift_right_logical(v, 1)
                par[b][pl.ds(g * 16, 16)] = lax.bitwise_and(v, 1) * D

        def gather_descs(b):
            return [
                pltpu.make_async_copy(s_hbm.at[qA[b]], gA[b], sem_g[b]),
                pltpu.make_async_copy(s_hbm.at[qB[b]], gB[b], sem_g[b]),
            ]

        def transpose_task(b):
            for half, gref in ((0, gA[b]), (1, gB[b])):
                def jg_body(jg, carry, gref=gref, half=half):
                    ja = half * 8 + jg
                    rowv = jg * 16 + iota16
                    parv = par[b][pl.ds(ja * 16, 16)]
                    for d in range(D):
                        x = plsc.load_gather(gref, [rowv, parv + d])
                        ts[b][d, pl.ds(ja * 16, 16)] = x
                    return carry
                lax.fori_loop(0, 8, jg_body, 0)

        def out_desc(j, b):
            h, b0 = task_pos(j)
            return pltpu.make_async_copy(
                ts[b], out_hbm.at[h, :, pl.ds(b0, TOK)], sem_o[b])

        def out_drain(b):
            pltpu.make_async_copy(
                ts[b], out_hbm.at[0, :, pl.ds(0, TOK)], sem_o[b]).wait()

        idx_desc(0, 0).start()
        idx_desc(0, 0).wait()
        compute_q(0)
        for dsc in gather_descs(0):
            dsc.start()
        idx_desc(1, 1).start()

        def pair_body(p, carry):
            for parity in range(2):
                j = 2 * p + 1 + parity
                bj = (1 + parity) % 2
                bp = 1 - bj

                @pl.when(j < PER)
                def _():
                    idx_desc(j, bj).wait()
                    compute_q(bj)
                    for dsc in gather_descs(bj):
                        dsc.start()

                    @pl.when(j + 1 < PER)
                    def _():
                        idx_desc(j + 1, bp).start()

                for dsc in gather_descs(bp):
                    dsc.wait()

                @pl.when(j - 1 >= 2)
                def _():
                    out_drain(bp)

                transpose_task(bp)
                out_desc(j - 1, bp).start()
            return carry

        lax.fori_loop(0, PER // 2, pair_body, 0)
        out_drain(0)
        out_drain(1)

    return phase2


def kernel(input, weight):
    bsz, hist = input.shape
    V, D = weight.shape
    wt = weight.T
    inp_t = input.T.astype(jnp.int32)
    rem = V % 256
    tail = lax.slice(weight, (V - rem, 0), (V, D)).reshape(rem // 2, 2 * D)
    S = _make_phase1(V, D)(wt, tail)
    out3 = _make_phase2(hist, bsz, V, D)(inp_t, S)
    return out3.transpose(2, 0, 1)


# speedup vs baseline: 1.2357x; 1.2357x over previous
import functools

import jax
import jax.numpy as jnp
from jax import lax
from jax.experimental import pallas as pl
from jax.experimental.pallas import tpu as pltpu
from jax.experimental.pallas import tpu_sc as plsc

_NC = 2
_NS = 16
_NW = _NC * _NS


def _mesh():
    return plsc.VectorSubcoreMesh(
        core_axis_name="c", subcore_axis_name="s",
        num_cores=_NC, num_subcores=_NS,
    )


@functools.lru_cache(maxsize=None)
def _make_phase1(V: int, D: int):
    CH = 256
    NFULL = V // CH
    REM = V - NFULL * CH
    PER = NFULL // _NW
    EXTRA = NFULL % _NW
    assert D == 64 and PER % 2 == 0 and REM in (0, 64) and V >= 2 * CH

    @functools.partial(
        pl.kernel,
        out_type=jax.ShapeDtypeStruct((V // 2, 2 * D), jnp.float32),
        mesh=_mesh(),
        scratch_types=[
            [pltpu.VMEM((D, CH), jnp.float32) for _ in range(2)],
            [pltpu.VMEM((CH // 2, 2 * D), jnp.float32) for _ in range(2)],
            [pltpu.SemaphoreType.DMA for _ in range(2)],
            [pltpu.SemaphoreType.DMA for _ in range(2)],
        ],
        compiler_params=pltpu.CompilerParams(needs_layout_passes=False),
    )
    def phase1(wt_hbm, tail_hbm, s_hbm, ins, outs, sem_in, sem_out):
        wid = lax.axis_index("s") * _NC + lax.axis_index("c")
        iota16 = lax.iota(jnp.int32, 16)

        def in_desc(c, b):
            return pltpu.make_async_copy(
                wt_hbm.at[:, pl.ds(c * CH, CH)], ins[b], sem_in[b])

        def out_desc(c, b):
            return pltpu.make_async_copy(
                outs[b], s_hbm.at[pl.ds(c * (CH // 2), CH // 2)], sem_out[b])

        def out_drain(b):
            pltpu.make_async_copy(
                outs[b], s_hbm.at[pl.ds(0, CH // 2)], sem_out[b]).wait()

        def transpose_slab(in_s, out_s, ntok):
            @plsc.parallel_loop(0, ntok // 16, unroll=4)
            def _tg(g):
                tb = g * 16
                tvec = tb + iota16
                row = lax.shift_right_logical(tvec, 1)
                colb = lax.bitwise_and(tvec, 1) * D
                for d in range(D):
                    x = in_s[d, pl.ds(tb, 16)]
                    plsc.store_scatter(out_s, [row, colb + d], x)

        nj = PER + jnp.where(wid < EXTRA, 1, 0)
        in_desc(wid, 0).start()

        def pair_body(p, carry):
            for parity in range(2):
                j = 2 * p + parity
                b = parity
                c = wid + _NW * j
                in_desc(c, b).wait()

                @pl.when(j + 1 < nj)
                def _():
                    in_desc(c + _NW, 1 - b).start()

                @pl.when(j >= 2)
                def _():
                    out_drain(b)

                transpose_slab(ins[b], outs[b], CH)
                out_desc(c, b).start()
            return carry

        lax.fori_loop(0, PER // 2, pair_body, 0)

        @pl.when(wid < EXTRA)
        def _():
            c = wid + _NW * PER
            in_desc(c, 0).wait()
            out_drain(0)
            transpose_slab(ins[0], outs[0], CH)
            out_desc(c, 0).start()

        if REM:
            @pl.when(wid == EXTRA)
            def _():
                out_drain(1)
                pltpu.make_async_copy(
                    tail_hbm, outs[1].at[pl.ds(0, REM // 2)],
                    sem_in[1]).start()
                pltpu.make_async_copy(
                    tail_hbm, outs[1].at[pl.ds(0, REM // 2)],
                    sem_in[1]).wait()
                pltpu.make_async_copy(
                    outs[1].at[pl.ds(0, REM // 2)],
                    s_hbm.at[pl.ds((V - REM) // 2, REM // 2)],
                    sem_out[1]).start()

        out_drain(0)
        if REM:
            @pl.when(wid == EXTRA)
            def _():
                pltpu.make_async_copy(
                    outs[1].at[pl.ds(0, REM // 2)],
                    s_hbm.at[pl.ds(0, REM // 2)], sem_out[1]).wait()

            @pl.when(wid != EXTRA)
            def _():
                out_drain(1)
        else:
            out_drain(1)

    return phase1


@functools.lru_cache(maxsize=None)
def _make_phase2(H: int, Bt: int, V: int, D: int):
    TOK = 256
    TPH = Bt // TOK
    TASKS = H * TPH
    PER = TASKS // _NW
    assert D == 64 and Bt % TOK == 0 and TASKS % _NW == 0 and PER % 2 == 0

    @functools.partial(
        pl.kernel,
        out_type=jax.ShapeDtypeStruct((H, D, Bt), jnp.float32),
        mesh=_mesh(),
        scratch_types=[
            [pltpu.VMEM((TOK,), jnp.int32) for _ in range(2)],
            [pltpu.VMEM((TOK,), jnp.int32) for _ in range(2)],
            [pltpu.VMEM((128,), jnp.int32) for _ in range(2)],
            [pltpu.VMEM((128,), jnp.int32) for _ in range(2)],
            [pltpu.VMEM((128, 2 * D), jnp.float32) for _ in range(2)],
            [pltpu.VMEM((128, 2 * D), jnp.float32) for _ in range(2)],
            [pltpu.VMEM((D, TOK), jnp.float32) for _ in range(2)],
            [pltpu.SemaphoreType.DMA for _ in range(2)],
            [pltpu.SemaphoreType.DMA for _ in range(2)],
            [pltpu.SemaphoreType.DMA for _ in range(2)],
        ],
        compiler_params=pltpu.CompilerParams(needs_layout_passes=False),
    )
    def phase2(inpt_hbm, s_hbm, out_hbm, idx, par, qA, qB, gA, gB, ts,
               sem_idx, sem_g, sem_o):
        wid = lax.axis_index("s") * _NC + lax.axis_index("c")
        iota16 = lax.iota(jnp.int32, 16)

        def task_pos(j):
            tau = wid + _NW * j
            return tau // TPH, (tau % TPH) * TOK

        def idx_desc(j, b):
            h, b0 = task_pos(j)
            return pltpu.make_async_copy(
                inpt_hbm.at[h, pl.ds(b0, TOK)], idx[b], sem_idx[b])

        def compute_q(b):
            for g in range(TOK // 16):
                v = idx[b][pl.ds(g * 16, 16)]
                half = qA if g < 8 else qB
                half[b][pl.ds((g % 8) * 16, 16)] = lax.shift_right_logical(v, 1)
                par[b][pl.ds(g * 16, 16)] = lax.bitwise_and(v, 1) * D

        def gather_descs(b):
            return [
                pltpu.make_async_copy(s_hbm.at[qA[b]], gA[b], sem_g[b]),
                pltpu.make_async_copy(s_hbm.at[qB[b]], gB[b], sem_g[b]),
            ]

        def transpose_task(b):
            for half, gref in ((0, gA[b]), (1, gB[b])):
                @plsc.parallel_loop(0, 8, unroll=4)
                def _jg(jg, gref=gref, half=half):
                    ja = half * 8 + jg
                    rowv = jg * 16 + iota16
                    parv = par[b][pl.ds(ja * 16, 16)]
                    for d in range(D):
                        x = plsc.load_gather(gref, [rowv, parv + d])
                        ts[b][d, pl.ds(ja * 16, 16)] = x

        def out_desc(j, b):
            h, b0 = task_pos(j)
            return pltpu.make_async_copy(
                ts[b], out_hbm.at[h, :, pl.ds(b0, TOK)], sem_o[b])

        def out_drain(b):
            pltpu.make_async_copy(
                ts[b], out_hbm.at[0, :, pl.ds(0, TOK)], sem_o[b]).wait()

        idx_desc(0, 0).start()
        idx_desc(0, 0).wait()
        compute_q(0)
        for dsc in gather_descs(0):
            dsc.start()
        idx_desc(1, 1).start()

        def pair_body(p, carry):
            for parity in range(2):
                j = 2 * p + 1 + parity
                bj = (1 + parity) % 2
                bp = 1 - bj

                @pl.when(j < PER)
                def _():
                    idx_desc(j, bj).wait()
                    compute_q(bj)
                    for dsc in gather_descs(bj):
                        dsc.start()

                    @pl.when(j + 1 < PER)
                    def _():
                        idx_desc(j + 1, bp).start()

                for dsc in gather_descs(bp):
                    dsc.wait()

                @pl.when(j - 1 >= 2)
                def _():
                    out_drain(bp)

                transpose_task(bp)
                out_desc(j - 1, bp).start()
            return carry

        lax.fori_loop(0, PER // 2, pair_body, 0)
        out_drain(0)
        out_drain(1)

    return phase2


def kernel(input, weight):
    bsz, hist = input.shape
    V, D = weight.shape
    wt = weight.T
    inp_t = input.T.astype(jnp.int32)
    rem = V % 256
    tail = lax.slice(weight, (V - rem, 0), (V, D)).reshape(rem // 2, 2 * D)
    S = _make_phase1(V, D)(wt, tail)
    out3 = _make_phase2(hist, bsz, V, D)(inp_t, S)
    return out3.transpose(2, 0, 1)


# speedup vs baseline: 1.4005x; 1.1334x over previous
import functools

import jax
import jax.numpy as jnp
from jax import lax
from jax.experimental import pallas as pl
from jax.experimental.pallas import tpu as pltpu
from jax.experimental.pallas import tpu_sc as plsc

_NC = 2
_NS = 16
_NW = _NC * _NS

_CPARAMS = pltpu.CompilerParams(
    needs_layout_passes=False, disable_bounds_checks=True)


def _mesh():
    return plsc.VectorSubcoreMesh(
        core_axis_name="c", subcore_axis_name="s",
        num_cores=_NC, num_subcores=_NS,
    )


@functools.lru_cache(maxsize=None)
def _make_phase1(V: int, D: int):
    CH = 256
    W = CH * D
    NFULL = V // CH
    REM = V - NFULL * CH
    PER = NFULL // _NW
    EXTRA = NFULL % _NW
    assert D == 64 and PER % 2 == 0 and REM in (0, 64) and V >= 2 * CH

    @functools.partial(
        pl.kernel,
        out_type=jax.ShapeDtypeStruct((V * D,), jnp.float32),
        mesh=_mesh(),
        scratch_types=[
            [pltpu.VMEM((D, CH), jnp.float32) for _ in range(2)],
            [pltpu.VMEM((W,), jnp.float32) for _ in range(2)],
            [pltpu.SemaphoreType.DMA for _ in range(2)],
            [pltpu.SemaphoreType.DMA for _ in range(2)],
        ],
        compiler_params=_CPARAMS,
    )
    def phase1(wt_hbm, tail_hbm, s_hbm, ins, outs, sem_in, sem_out):
        wid = lax.axis_index("s") * _NC + lax.axis_index("c")
        iota64 = lax.iota(jnp.int32, 16) * D

        def in_desc(c, b):
            return pltpu.make_async_copy(
                wt_hbm.at[:, pl.ds(c * CH, CH)], ins[b], sem_in[b])

        def out_desc(c, b):
            return pltpu.make_async_copy(
                outs[b], s_hbm.at[pl.ds(c * W, W)], sem_out[b])

        def out_drain(b):
            pltpu.make_async_copy(
                outs[b], s_hbm.at[pl.ds(0, W)], sem_out[b]).wait()

        def transpose_slab(in_s, out_s, ntok):
            LA = 8

            @plsc.parallel_loop(0, ntok // 16, unroll=2)
            def _tg(g):
                tb = g * 16
                dst = out_s.at[pl.ds(tb * D, 16 * D)]
                xs = {}
                for d in range(LA):
                    xs[d] = in_s[d, pl.ds(tb, 16)]
                for d in range(D):
                    if d + LA < D:
                        xs[d + LA] = in_s[d + LA, pl.ds(tb, 16)]
                    plsc.store_scatter(
                        dst, [lax.bitwise_or(iota64, d)], xs.pop(d))

        nj = PER + jnp.where(wid < EXTRA, 1, 0)
        in_desc(wid, 0).start()

        def pair_body(p, carry):
            for parity in range(2):
                j = 2 * p + parity
                b = parity
                c = wid + _NW * j
                in_desc(c, b).wait()

                @pl.when(j + 1 < nj)
                def _():
                    in_desc(c + _NW, 1 - b).start()

                @pl.when(j >= 2)
                def _():
                    out_drain(b)

                transpose_slab(ins[b], outs[b], CH)
                out_desc(c, b).start()
            return carry

        lax.fori_loop(0, PER // 2, pair_body, 0)

        @pl.when(wid < EXTRA)
        def _():
            c = wid + _NW * PER
            in_desc(c, 0).wait()
            out_drain(0)
            transpose_slab(ins[0], outs[0], CH)
            out_desc(c, 0).start()

        if REM:
            TW = REM * D
            @pl.when(wid == EXTRA)
            def _():
                out_drain(1)
                pltpu.make_async_copy(
                    tail_hbm, outs[1].at[pl.ds(0, TW)], sem_in[1]).start()
                pltpu.make_async_copy(
                    tail_hbm, outs[1].at[pl.ds(0, TW)], sem_in[1]).wait()
                pltpu.make_async_copy(
                    outs[1].at[pl.ds(0, TW)],
                    s_hbm.at[pl.ds((V - REM) * D, TW)], sem_out[1]).start()

        out_drain(0)
        if REM:
            @pl.when(wid == EXTRA)
            def _():
                pltpu.make_async_copy(
                    outs[1].at[pl.ds(0, REM * D)],
                    s_hbm.at[pl.ds(0, REM * D)], sem_out[1]).wait()

            @pl.when(wid != EXTRA)
            def _():
                out_drain(1)
        else:
            out_drain(1)

    return phase1


@functools.lru_cache(maxsize=None)
def _make_phase2(H: int, Bt: int, V: int, D: int):
    TOK = 256
    TPH = Bt // TOK
    TASKS = H * TPH
    PER = TASKS // _NW
    assert D == 64 and Bt % TOK == 0 and TASKS % _NW == 0 and PER % 2 == 0

    @functools.partial(
        pl.kernel,
        out_type=jax.ShapeDtypeStruct((H, D, Bt), jnp.float32),
        mesh=_mesh(),
        scratch_types=[
            [pltpu.VMEM((TOK,), jnp.int32) for _ in range(2)],
            [pltpu.VMEM((TOK + 16,), jnp.int32) for _ in range(2)],
            [pltpu.VMEM((128,), jnp.int32) for _ in range(2)],
            [pltpu.VMEM((128,), jnp.int32) for _ in range(2)],
            [pltpu.VMEM((128, 2 * D), jnp.float32) for _ in range(2)],
            [pltpu.VMEM((128, 2 * D), jnp.float32) for _ in range(2)],
            [pltpu.VMEM((D * TOK,), jnp.float32) for _ in range(2)],
            [pltpu.SemaphoreType.DMA for _ in range(2)],
            [pltpu.SemaphoreType.DMA for _ in range(2)],
            [pltpu.SemaphoreType.DMA for _ in range(2)],
        ],
        compiler_params=_CPARAMS,
    )
    def phase2(inpt_hbm, s_hbm, out_hbm, idx, par, qA, qB, gA, gB, ts,
               sem_idx, sem_g, sem_o):
        wid = lax.axis_index("s") * _NC + lax.axis_index("c")
        iota16 = lax.iota(jnp.int32, 16)
        qvecs = [(q * 16 + iota16) * TOK for q in range(4)]

        def task_pos(j):
            tau = wid + _NW * j
            return tau // TPH, (tau % TPH) * TOK

        def idx_desc(j, b):
            h, b0 = task_pos(j)
            return pltpu.make_async_copy(
                inpt_hbm.at[h, pl.ds(b0, TOK)], idx[b], sem_idx[b])

        def compute_q(b):
            for g in range(TOK // 16):
                v = idx[b][pl.ds(g * 16, 16)]
                half = qA if g < 8 else qB
                half[b][pl.ds((g % 8) * 16, 16)] = lax.shift_right_logical(v, 1)
                par[b][pl.ds(g * 16, 16)] = lax.bitwise_and(v, 1) * D

        def gather_descs(b):
            return [
                pltpu.make_async_copy(s_hbm.at[qA[b]], gA[b], sem_g[b]),
                pltpu.make_async_copy(s_hbm.at[qB[b]], gB[b], sem_g[b]),
            ]

        def transpose_task(b):
            for half, gref in ((0, gA[b]), (1, gB[b])):
                @plsc.parallel_loop(0, 128, unroll=4)
                def _tok(r, gref=gref, half=half):
                    j = half * 128 + r
                    parv = par[b][pl.ds(j, 16)][0]
                    for q in range(4):
                        x = gref[r, pl.ds(parv + q * 16, 16)]
                        plsc.store_scatter(ts[b], [qvecs[q] + j], x)

        def out_starts(j, b):
            h, b0 = task_pos(j)
            for d in range(D):
                pltpu.make_async_copy(
                    ts[b].at[pl.ds(d * TOK, TOK)],
                    out_hbm.at[h, d, pl.ds(b0, TOK)], sem_o[b]).start()

        def out_drain(b):
            for d in range(D):
                pltpu.make_async_copy(
                    ts[b].at[pl.ds(0, TOK)],
                    out_hbm.at[0, 0, pl.ds(0, TOK)], sem_o[b]).wait()

        idx_desc(0, 0).start()
        idx_desc(0, 0).wait()
        compute_q(0)
        for dsc in gather_descs(0):
            dsc.start()
        idx_desc(1, 1).start()

        def pair_body(p, carry):
            for parity in range(2):
                j = 2 * p + 1 + parity
                bj = (1 + parity) % 2
                bp = 1 - bj

                @pl.when(j < PER)
                def _():
                    idx_desc(j, bj).wait()
                    compute_q(bj)
                    for dsc in gather_descs(bj):
                        dsc.start()

                    @pl.when(j + 1 < PER)
                    def _():
                        idx_desc(j + 1, bp).start()

                for dsc in gather_descs(bp):
                    dsc.wait()

                @pl.when(j - 1 >= 2)
                def _():
                    out_drain(bp)

                transpose_task(bp)
                out_starts(j - 1, bp)
            return carry

        lax.fori_loop(0, PER // 2, pair_body, 0)
        out_drain(0)
        out_drain(1)

    return phase2


def kernel(input, weight):
    bsz, hist = input.shape
    V, D = weight.shape
    wt = weight.T
    inp_t = input.T.astype(jnp.int32)
    rem = V % 256
    tail = lax.slice(weight, (V - rem, 0), (V, D)).reshape(rem * D)
    S = _make_phase1(V, D)(wt, tail)
    S2 = S.reshape(V // 2, 2 * D)
    out3 = _make_phase2(hist, bsz, V, D)(inp_t, S2)
    return out3.transpose(2, 0, 1)


# speedup vs baseline: 1.9644x; 1.4026x over previous
import functools

import jax
import jax.numpy as jnp
from jax import lax
from jax.experimental import pallas as pl
from jax.experimental.pallas import tpu as pltpu
from jax.experimental.pallas import tpu_sc as plsc

_NC = 2
_NS = 16
_NW = _NC * _NS

_CHUNK = 128
_SUP = 2
_ROWS = _CHUNK * _SUP
_NBUF = 4


@functools.lru_cache(maxsize=None)
def _make_gather(B: int, V: int, D: int):
    assert B % (_NW * _ROWS * _NBUF) == 0
    b_per_w = B // _NW
    n_sup = b_per_w // _ROWS
    assert n_sup % _NBUF == 0 and n_sup >= 2 * _NBUF

    mesh = plsc.VectorSubcoreMesh(
        core_axis_name="c", subcore_axis_name="s",
        num_cores=_NC, num_subcores=_NS,
    )

    @functools.partial(
        pl.kernel,
        out_type=jax.ShapeDtypeStruct((B, D), jnp.float32),
        mesh=mesh,
        scratch_types=[
            pltpu.VMEM((b_per_w,), jnp.int32),
            [pltpu.VMEM((_ROWS, D), jnp.float32) for _ in range(_NBUF)],
            [pltpu.SemaphoreType.DMA for _ in range(_NBUF)],
            [pltpu.SemaphoreType.DMA for _ in range(_NBUF)],
        ],
        compiler_params=pltpu.CompilerParams(use_tc_tiling_on_sc=False),
    )
    def gather_kernel(idx_hbm, table_hbm, out_hbm, idx_v, rows, sem_g, sem_o):
        wid = lax.axis_index("s") * _NC + lax.axis_index("c")
        base = wid * b_per_w

        pltpu.sync_copy(idx_hbm.at[pl.ds(base, b_per_w)], idx_v)

        def gather_descs(s, b):
            return [
                pltpu.make_async_copy(
                    table_hbm.at[idx_v.at[pl.ds(s * _ROWS + c * _CHUNK, _CHUNK)]],
                    rows[b].at[pl.ds(c * _CHUNK, _CHUNK)],
                    sem_g[b],
                )
                for c in range(_SUP)
            ]

        def out_desc(s, b):
            return pltpu.make_async_copy(
                rows[b], out_hbm.at[pl.ds(base + s * _ROWS, _ROWS)], sem_o[b]
            )

        def issue_gathers(s, b):
            for d in gather_descs(s, b):
                d.start()

        def wait_gathers(s, b):
            for d in gather_descs(s, b):
                d.wait()

        issue_gathers(0, 0)
        issue_gathers(1, 1)
        wait_gathers(0, 0)
        out_desc(0, 0).start()
        issue_gathers(2, 2)
        wait_gathers(1, 1)
        out_desc(1, 1).start()
        issue_gathers(3, 3)

        def block(g, carry):
            for b in range(_NBUF):
                s = _NBUF + g * _NBUF + b
                out_desc(s - _NBUF, b).wait()
                issue_gathers(s, b)
                bq = (b + _NBUF - 2) % _NBUF
                wait_gathers(s - 2, bq)
                out_desc(s - 2, bq).start()
            return carry

        lax.fori_loop(0, n_sup // _NBUF - 1, block, 0)

        for s in (n_sup - 2, n_sup - 1):
            b = s % _NBUF
            wait_gathers(s, b)
            out_desc(s, b).start()
        for s in range(n_sup - _NBUF, n_sup):
            out_desc(s, s % _NBUF).wait()

    return gather_kernel


def kernel(input, weight):
    bsz, hist = input.shape
    V, D = weight.shape
    idx = input.reshape(-1).astype(jnp.int32)
    out = _make_gather(idx.shape[0], V, D)(idx, weight)
    return out.reshape(bsz, hist, D)
